# baseline (speedup 1.0000x reference)
import functools

import jax
import jax.numpy as jnp
from jax import lax
from jax.experimental import pallas as pl
from jax.experimental.pallas import tpu as pltpu
from jax.experimental.pallas import tpu_sc as plsc

QS = 16384
NB_ROWS = 4096
ROW = 16 * 8 * 8
TAIL = QS - NB_ROWS
NW = 32
RPW = QS // NW
NBW = NB_ROWS // RPW
CH = 32
NCH = RPW // CH


NBUF = 3


def _ring_copy(src, dst, base, bufs, in_sem, out_sem):
    def in_copy(c):
        k = c % NBUF
        return pltpu.make_async_copy(
            src.at[pl.ds(base + c * CH, CH)], bufs[k], in_sem.at[k]
        )

    def out_copy(c):
        k = c % NBUF
        return pltpu.make_async_copy(
            bufs[k], dst.at[pl.ds(base + c * CH, CH)], out_sem.at[k]
        )

    for c in range(NBUF - 1):
        in_copy(c).start()
    for c in range(NCH):
        if c + NBUF - 1 < NCH:
            if c >= 1:
                out_copy(c - 1).wait()
            in_copy(c + NBUF - 1).start()
        in_copy(c).wait()
        out_copy(c).start()
    for c in range(NCH - NBUF, NCH):
        out_copy(c).wait()


def _sc_store(bf, f, bl, lab, out, lab_out, s0, s1, s2, in_sem, out_sem, lsem):
    sid = lax.axis_index("s")
    wid = sid * 2 + lax.axis_index("c")
    base = wid * RPW
    bufs = (s0.at[sid], s1.at[sid], s2.at[sid])

    @pl.when(wid < NBW)
    def _():
        _ring_copy(bf, out, base, bufs, in_sem, out_sem)

    @pl.when(wid >= NBW)
    def _():
        _ring_copy(f, out, base, bufs, in_sem, out_sem)

    @pl.when(wid == 0)
    def _():
        pltpu.async_copy(bl, lab_out.at[pl.ds(0, NB_ROWS)], lsem).wait()

    @pl.when(wid == 1)
    def _():
        pltpu.async_copy(
            lab.at[pl.ds(NB_ROWS, TAIL)], lab_out.at[pl.ds(NB_ROWS, TAIL)], lsem
        ).wait()


def kernel(batch_features, batch_labels, features, labels):
    bf = batch_features.reshape(NB_ROWS, ROW)
    f = features.reshape(QS, ROW)
    mesh = plsc.VectorSubcoreMesh(core_axis_name="c", subcore_axis_name="s")
    run = functools.partial(
        pl.kernel,
        _sc_store,
        out_type=[
            jax.ShapeDtypeStruct((QS, ROW), jnp.float32),
            jax.ShapeDtypeStruct((QS,), jnp.int32),
        ],
        mesh=mesh,
        scratch_types=[
            pltpu.MemorySpace.VMEM_SHARED((16, CH, ROW), jnp.float32),
            pltpu.MemorySpace.VMEM_SHARED((16, CH, ROW), jnp.float32),
            pltpu.MemorySpace.VMEM_SHARED((16, CH, ROW), jnp.float32),
            pltpu.SemaphoreType.DMA((NBUF,)),
            pltpu.SemaphoreType.DMA((NBUF,)),
            pltpu.SemaphoreType.DMA,
        ],
    )()
    out, lab_out = run(bf, f, batch_labels, labels)
    return out.reshape(QS, 16, 8, 8), lab_out


# speedup vs baseline: 1.0423x; 1.0423x over previous
import functools

import jax
import jax.numpy as jnp
from jax import lax
from jax.experimental import pallas as pl
from jax.experimental.pallas import tpu as pltpu
from jax.experimental.pallas import tpu_sc as plsc

QS = 16384
NB_ROWS = 4096
ROW = 16 * 8 * 8
TAIL = QS - NB_ROWS
NW = 32
RPW = NB_ROWS // NW
CH = 32
NCH = RPW // CH
NBUF = 3
BLK = 2048
TGRID = TAIL // BLK
TOFF = NB_ROWS // BLK


def _ring_copy(src, dst, base, bufs, in_sem, out_sem):
    def in_copy(c):
        k = c % NBUF
        return pltpu.make_async_copy(
            src.at[pl.ds(base + c * CH, CH)], bufs[k], in_sem.at[k]
        )

    def out_copy(c):
        k = c % NBUF
        return pltpu.make_async_copy(
            bufs[k], dst.at[pl.ds(base + c * CH, CH)], out_sem.at[k]
        )

    for c in range(min(NBUF - 1, NCH)):
        in_copy(c).start()
    for c in range(NCH):
        if c + NBUF - 1 < NCH:
            if c >= 1:
                out_copy(c - 1).wait()
            in_copy(c + NBUF - 1).start()
        in_copy(c).wait()
        out_copy(c).start()
    for c in range(max(NCH - NBUF, 0), NCH):
        out_copy(c).wait()


def _sc_scatter_head(bf, bl, lab, out, lab_out, s0, s1, s2, in_sem, out_sem, lsem):
    sid = lax.axis_index("s")
    wid = sid * 2 + lax.axis_index("c")
    base = wid * RPW
    bufs = (s0.at[sid], s1.at[sid], s2.at[sid])

    _ring_copy(bf, out, base, bufs, in_sem, out_sem)

    @pl.when(wid == 0)
    def _():
        pltpu.async_copy(bl, lab_out.at[pl.ds(0, NB_ROWS)], lsem).wait()

    @pl.when(wid == 1)
    def _():
        pltpu.async_copy(
            lab.at[pl.ds(NB_ROWS, TAIL)], lab_out.at[pl.ds(NB_ROWS, TAIL)], lsem
        ).wait()


def _tc_tail_copy(f_ref, head_ref, out_ref):
    del head_ref
    out_ref[...] = f_ref[...]


def kernel(batch_features, batch_labels, features, labels):
    bf = batch_features.reshape(NB_ROWS, ROW)
    f = features.reshape(QS, ROW)
    mesh = plsc.VectorSubcoreMesh(core_axis_name="c", subcore_axis_name="s")
    sc_run = functools.partial(
        pl.kernel,
        _sc_scatter_head,
        out_type=[
            jax.ShapeDtypeStruct((QS, ROW), jnp.float32),
            jax.ShapeDtypeStruct((QS,), jnp.int32),
        ],
        mesh=mesh,
        scratch_types=[
            pltpu.MemorySpace.VMEM_SHARED((16, CH, ROW), jnp.float32),
            pltpu.MemorySpace.VMEM_SHARED((16, CH, ROW), jnp.float32),
            pltpu.MemorySpace.VMEM_SHARED((16, CH, ROW), jnp.float32),
            pltpu.SemaphoreType.DMA((NBUF,)),
            pltpu.SemaphoreType.DMA((NBUF,)),
            pltpu.SemaphoreType.DMA,
        ],
    )()
    out_head, lab_out = sc_run(bf, batch_labels, labels)
    out = pl.pallas_call(
        _tc_tail_copy,
        grid=(TGRID,),
        in_specs=[
            pl.BlockSpec((BLK, ROW), lambda i: (i + TOFF, 0)),
            pl.BlockSpec(memory_space=pltpu.MemorySpace.HBM),
        ],
        out_specs=pl.BlockSpec((BLK, ROW), lambda i: (i + TOFF, 0)),
        out_shape=jax.ShapeDtypeStruct((QS, ROW), jnp.float32),
        input_output_aliases={1: 0},
    )(f, out_head)
    return out.reshape(QS, 16, 8, 8), lab_out
